# Initial kernel scaffold; baseline (speedup 1.0000x reference)
import jax
import jax.numpy as jnp
from jax.experimental import pallas as pl


def kernel(x, W1, b1, g1, be1, W2, b2, g2, be2, edge_index):
    raise NotImplementedError("write your pallas kernel here")


# speedup vs baseline: 12.6249x; 12.6249x over previous
import functools

import jax
import jax.numpy as jnp
from jax import lax
from jax.experimental import pallas as pl
from jax.experimental.pallas import tpu as pltpu
from jax.experimental.pallas import tpu_sc as plsc

N = 10000
E = 320000
D = 128
EPS = 1e-5

W = 128
NWIN = E // W
NCORES = 2
NSUB = 16
NWORK = NCORES * NSUB
RCHUNK = 80
NCHUNK = N // RCHUNK

_MESH = plsc.VectorSubcoreMesh(core_axis_name="c", subcore_axis_name="s")


@functools.partial(
    pl.kernel,
    out_type=jax.ShapeDtypeStruct((N,), jnp.float32),
    mesh=_MESH,
    scratch_types=[
        pltpu.VMEM_SHARED((N,), jnp.float32),
        pltpu.VMEM((N,), jnp.float32),
        pltpu.VMEM((W,), jnp.float32),
        pltpu.VMEM((W,), jnp.int32),
    ],
)
def _sc_degree(col_hbm, zeros_hbm, ones_hbm, deg_hbm, dacc, zbuf, ones_v, cidx):
    c = lax.axis_index("c")
    s = lax.axis_index("s")

    @pl.when(c == 0)
    def _():
        @pl.when(s == 0)
        def _():
            pltpu.sync_copy(zeros_hbm, zbuf)
            pltpu.sync_copy(zbuf, dacc)
        pltpu.sync_copy(ones_hbm, ones_v)
        plsc.subcore_barrier()

        nfull = NWIN // NSUB

        @pl.loop(0, nfull)
        def _(t):
            eb = (s + t * NSUB) * W
            pltpu.sync_copy(col_hbm.at[pl.ds(eb, W)], cidx)
            pltpu.sync_copy(ones_v, dacc.at[cidx], add=True)

        @pl.when(s < NWIN - nfull * NSUB)
        def _():
            eb = (nfull * NSUB + s) * W
            pltpu.sync_copy(col_hbm.at[pl.ds(eb, W)], cidx)
            pltpu.sync_copy(ones_v, dacc.at[cidx], add=True)

        plsc.subcore_barrier()

        @pl.when(s == 0)
        def _():
            pltpu.sync_copy(dacc, deg_hbm)


@functools.partial(
    pl.kernel,
    out_type=jax.ShapeDtypeStruct((NCORES, N, D), jnp.float32),
    mesh=_MESH,
    scratch_types=[
        pltpu.VMEM_SHARED((N, D), jnp.float32),
        pltpu.VMEM((W, D), jnp.float32),
        pltpu.VMEM((W,), jnp.int32),
        pltpu.VMEM((W,), jnp.int32),
    ],
)
def _sc_mp(h_hbm, row_hbm, col_hbm, zeros_hbm, out_hbm, acc, rows_v, cidx, ridx):
    c = lax.axis_index("c")
    s = lax.axis_index("s")
    w = c * NSUB + s

    pltpu.sync_copy(zeros_hbm, rows_v)

    @pl.loop(0, (NCHUNK + NSUB - 1) // NSUB)
    def _(t):
        ck = s + t * NSUB

        @pl.when(ck < NCHUNK)
        def _():
            pltpu.sync_copy(rows_v.at[pl.ds(0, RCHUNK)],
                            acc.at[pl.ds(ck * RCHUNK, RCHUNK)])

    plsc.subcore_barrier()

    nfull = NWIN // NWORK

    @pl.loop(0, nfull)
    def _(t):
        eb = (w + t * NWORK) * W
        pltpu.sync_copy(col_hbm.at[pl.ds(eb, W)], cidx)
        pltpu.sync_copy(h_hbm.at[cidx], rows_v)
        pltpu.sync_copy(row_hbm.at[pl.ds(eb, W)], ridx)
        pltpu.sync_copy(rows_v, acc.at[ridx], add=True)

    @pl.when(w < NWIN - nfull * NWORK)
    def _():
        eb = (nfull * NWORK + w) * W
        pltpu.sync_copy(col_hbm.at[pl.ds(eb, W)], cidx)
        pltpu.sync_copy(h_hbm.at[cidx], rows_v)
        pltpu.sync_copy(row_hbm.at[pl.ds(eb, W)], ridx)
        pltpu.sync_copy(rows_v, acc.at[ridx], add=True)

    plsc.subcore_barrier()

    @pl.loop(0, (NCHUNK + NSUB - 1) // NSUB)
    def _(t):
        ck = s + t * NSUB

        @pl.when(ck < NCHUNK)
        def _():
            pltpu.sync_copy(acc.at[pl.ds(ck * RCHUNK, RCHUNK)],
                            out_hbm.at[c, pl.ds(ck * RCHUNK, RCHUNK)])


_BLK = 2000


def _bn_gelu(h, g, be):
    mu = jnp.mean(h, axis=1, keepdims=True)
    var = jnp.mean((h - mu) ** 2, axis=1, keepdims=True)
    h = (h - mu) * lax.rsqrt(var + EPS)
    h = h * g + be
    return 0.5 * h * (1.0 + lax.erf(h * (2.0 ** -0.5)))


def _tc_dense1_body(x_ref, wt_ref, b_ref, g_ref, be_ref, o_ref):
    h = lax.dot_general(x_ref[...], wt_ref[...], (((1,), (0,)), ((), ())),
                        precision=lax.Precision.HIGHEST,
                        preferred_element_type=jnp.float32)
    h = h + b_ref[...]
    o_ref[...] = _bn_gelu(h, g_ref[...], be_ref[...])


def _tc_scale_body(h_ref, deg_ref, hp_ref, dis_ref):
    dis = lax.rsqrt(deg_ref[...])
    dis_ref[...] = dis
    hp_ref[...] = h_ref[...] * dis


def _tc_layer2_body(acc_ref, dis_ref, wt_ref, b_ref, g_ref, be_ref, o_ref):
    dis = dis_ref[...]
    a = (acc_ref[0] + acc_ref[1]) * dis
    h = lax.dot_general(a, wt_ref[...], (((1,), (0,)), ((), ())),
                        precision=lax.Precision.HIGHEST,
                        preferred_element_type=jnp.float32)
    h = h + b_ref[...]
    o_ref[...] = _bn_gelu(h, g_ref[...], be_ref[...]) * dis


def _tc_final_body(acc_ref, dis_ref, o_ref):
    o_ref[...] = (acc_ref[0] + acc_ref[1]) * dis_ref[...]


def _row_spec(r, cdim):
    return pl.BlockSpec((r, cdim), lambda i: (i, 0))


def _full_spec(shape):
    return pl.BlockSpec(shape, lambda i: tuple(0 for _ in shape))


def _tc_dense1(x, w1t, b1, g1, be1):
    return pl.pallas_call(
        _tc_dense1_body,
        grid=(N // _BLK,),
        in_specs=[_row_spec(_BLK, D), _full_spec((D, D)), _full_spec((1, D)),
                  _row_spec(_BLK, 1), _row_spec(_BLK, 1)],
        out_specs=_row_spec(_BLK, D),
        out_shape=jax.ShapeDtypeStruct((N, D), jnp.float32),
    )(x, w1t, b1, g1, be1)


def _tc_scale(h, deg):
    return pl.pallas_call(
        _tc_scale_body,
        grid=(N // _BLK,),
        in_specs=[_row_spec(_BLK, D), _row_spec(_BLK, 1)],
        out_specs=[_row_spec(_BLK, D), _row_spec(_BLK, 1)],
        out_shape=[jax.ShapeDtypeStruct((N, D), jnp.float32),
                   jax.ShapeDtypeStruct((N, 1), jnp.float32)],
    )(h, deg)


def _tc_layer2(accp, dis, w2t, b2, g2, be2):
    return pl.pallas_call(
        _tc_layer2_body,
        grid=(N // _BLK,),
        in_specs=[pl.BlockSpec((NCORES, _BLK, D), lambda i: (0, i, 0)),
                  _row_spec(_BLK, 1), _full_spec((D, D)), _full_spec((1, D)),
                  _row_spec(_BLK, 1), _row_spec(_BLK, 1)],
        out_specs=_row_spec(_BLK, D),
        out_shape=jax.ShapeDtypeStruct((N, D), jnp.float32),
    )(accp, dis, w2t, b2, g2, be2)


def _tc_final(accp, dis):
    return pl.pallas_call(
        _tc_final_body,
        grid=(N // _BLK,),
        in_specs=[pl.BlockSpec((NCORES, _BLK, D), lambda i: (0, i, 0)),
                  _row_spec(_BLK, 1)],
        out_specs=_row_spec(_BLK, D),
        out_shape=jax.ShapeDtypeStruct((N, D), jnp.float32),
    )(accp, dis)


def kernel(x, W1, b1, g1, be1, W2, b2, g2, be2, edge_index):
    xs = x[0]
    row = edge_index[0]
    col = edge_index[1]
    zeros2d = jnp.zeros((W, D), jnp.float32)
    zeros1d = jnp.zeros((N,), jnp.float32)
    ones1d = jnp.ones((W,), jnp.float32)

    deg = _sc_degree(col, zeros1d, ones1d)
    h1 = _tc_dense1(xs, W1.T, b1.reshape(1, D), g1.reshape(N, 1),
                    be1.reshape(N, 1))
    h1p, dis = _tc_scale(h1, deg.reshape(N, 1))
    accp1 = _sc_mp(h1p, row, col, zeros2d)
    h2p = _tc_layer2(accp1, dis, W2.T, b2.reshape(1, D), g2.reshape(N, 1),
                     be2.reshape(N, 1))
    accp2 = _sc_mp(h2p, row, col, zeros2d)
    out = _tc_final(accp2, dis)
    return out[None]


# speedup vs baseline: 20.4863x; 1.6227x over previous
import functools

import jax
import jax.numpy as jnp
from jax import lax
from jax.experimental import pallas as pl
from jax.experimental.pallas import tpu as pltpu
from jax.experimental.pallas import tpu_sc as plsc

N = 10000
E = 320000
D = 128
EPS = 1e-5

W = 128
NWIN = E // W
NCORES = 2
NSUB = 16
NWORK = NCORES * NSUB
RCHUNK = 80
NCHUNK = N // RCHUNK

_MESH = plsc.VectorSubcoreMesh(core_axis_name="c", subcore_axis_name="s")


@functools.partial(
    pl.kernel,
    out_type=jax.ShapeDtypeStruct((NCORES, N), jnp.float32),
    mesh=_MESH,
    scratch_types=[
        pltpu.VMEM_SHARED((N,), jnp.float32),
        pltpu.VMEM((N,), jnp.float32),
        pltpu.VMEM((W,), jnp.float32),
        pltpu.VMEM((2, W), jnp.int32),
        pltpu.SemaphoreType.DMA,
        pltpu.SemaphoreType.DMA,
    ],
)
def _sc_degree(col_hbm, zeros_hbm, ones_hbm, deg_hbm, dacc, zbuf, ones_v,
               cidx2, isem0, isem1):
    c = lax.axis_index("c")
    s = lax.axis_index("s")
    w = c * NSUB + s

    @pl.when(s == 0)
    def _():
        pltpu.sync_copy(zeros_hbm, zbuf)
        pltpu.sync_copy(zbuf, dacc)
    pltpu.sync_copy(ones_hbm, ones_v)
    plsc.subcore_barrier()

    nfull = NWIN // NWORK
    npair = nfull // 2

    def eb(j):
        return (w + j * NWORK) * W

    pltpu.sync_copy(col_hbm.at[pl.ds(eb(0), W)], cidx2.at[0])

    @pl.loop(0, npair)
    def _(r):
        pltpu.async_copy(col_hbm.at[pl.ds(eb(2 * r + 1), W)], cidx2.at[1],
                         isem1)
        pltpu.sync_copy(ones_v, dacc.at[cidx2.at[0]], add=True)

        @pl.when(r < npair - 1)
        def _():
            pltpu.async_copy(col_hbm.at[pl.ds(eb(2 * r + 2), W)], cidx2.at[0],
                             isem0)

        pltpu.make_async_copy(col_hbm.at[pl.ds(0, W)], cidx2.at[1],
                              isem1).wait()
        pltpu.sync_copy(ones_v, dacc.at[cidx2.at[1]], add=True)

        @pl.when(r < npair - 1)
        def _():
            pltpu.make_async_copy(col_hbm.at[pl.ds(0, W)], cidx2.at[0],
                                  isem0).wait()

    @pl.when(w < NWIN - nfull * NWORK)
    def _():
        pltpu.sync_copy(col_hbm.at[pl.ds((nfull * NWORK + w) * W, W)],
                        cidx2.at[0])
        pltpu.sync_copy(ones_v, dacc.at[cidx2.at[0]], add=True)

    plsc.subcore_barrier()

    @pl.when(s == 0)
    def _():
        pltpu.sync_copy(dacc, deg_hbm.at[c])


@functools.partial(
    pl.kernel,
    out_type=jax.ShapeDtypeStruct((NCORES, N, D), jnp.float32),
    mesh=_MESH,
    scratch_types=[
        pltpu.VMEM_SHARED((N, D), jnp.float32),
        pltpu.VMEM((2, W, D), jnp.float32),
        pltpu.VMEM((2, W), jnp.int32),
        pltpu.VMEM((2, W), jnp.int32),
        pltpu.SemaphoreType.DMA,
        pltpu.SemaphoreType.DMA,
    ],
)
def _sc_mp(h_hbm, row_hbm, col_hbm, zeros_hbm, out_hbm, acc, rows2, cidx2,
           ridx2, gsem0, gsem1):
    c = lax.axis_index("c")
    s = lax.axis_index("s")
    w = c * NSUB + s

    pltpu.sync_copy(zeros_hbm, rows2.at[0])

    @pl.loop(0, (NCHUNK + NSUB - 1) // NSUB)
    def _(t):
        ck = s + t * NSUB

        @pl.when(ck < NCHUNK)
        def _():
            pltpu.sync_copy(rows2.at[0, pl.ds(0, RCHUNK)],
                            acc.at[pl.ds(ck * RCHUNK, RCHUNK)])

    plsc.subcore_barrier()

    nfull = NWIN // NWORK
    npair = nfull // 2

    def eb(j):
        return (w + j * NWORK) * W

    def load_idx(j, b):
        pltpu.sync_copy(col_hbm.at[pl.ds(eb(j), W)], cidx2.at[b])
        pltpu.sync_copy(row_hbm.at[pl.ds(eb(j), W)], ridx2.at[b])

    load_idx(0, 0)
    pltpu.async_copy(h_hbm.at[cidx2.at[0]], rows2.at[0], gsem0)

    @pl.loop(0, npair)
    def _(r):
        load_idx(2 * r + 1, 1)
        g1 = pltpu.async_copy(h_hbm.at[cidx2.at[1]], rows2.at[1], gsem1)
        pltpu.make_async_copy(h_hbm.at[cidx2.at[0]], rows2.at[0],
                              gsem0).wait()
        pltpu.sync_copy(rows2.at[0], acc.at[ridx2.at[0]], add=True)

        @pl.when(r < npair - 1)
        def _():
            load_idx(2 * r + 2, 0)
            pltpu.async_copy(h_hbm.at[cidx2.at[0]], rows2.at[0], gsem0)

        g1.wait()
        pltpu.sync_copy(rows2.at[1], acc.at[ridx2.at[1]], add=True)

    @pl.when(w < NWIN - nfull * NWORK)
    def _():
        j = nfull * NWORK + w
        pltpu.sync_copy(col_hbm.at[pl.ds(j * W, W)], cidx2.at[0])
        pltpu.sync_copy(row_hbm.at[pl.ds(j * W, W)], ridx2.at[0])
        pltpu.sync_copy(h_hbm.at[cidx2.at[0]], rows2.at[0])
        pltpu.sync_copy(rows2.at[0], acc.at[ridx2.at[0]], add=True)

    plsc.subcore_barrier()

    @pl.loop(0, (NCHUNK + NSUB - 1) // NSUB)
    def _(t):
        ck = s + t * NSUB

        @pl.when(ck < NCHUNK)
        def _():
            pltpu.sync_copy(acc.at[pl.ds(ck * RCHUNK, RCHUNK)],
                            out_hbm.at[c, pl.ds(ck * RCHUNK, RCHUNK)])


_BLK = 2000


def _bn_gelu(h, g, be):
    mu = jnp.mean(h, axis=1, keepdims=True)
    var = jnp.mean((h - mu) ** 2, axis=1, keepdims=True)
    h = (h - mu) * lax.rsqrt(var + EPS)
    h = h * g + be
    return 0.5 * h * (1.0 + lax.erf(h * (2.0 ** -0.5)))


def _tc_dense1_body(x_ref, wt_ref, b_ref, g_ref, be_ref, o_ref):
    h = lax.dot_general(x_ref[...], wt_ref[...], (((1,), (0,)), ((), ())),
                        precision=lax.Precision.HIGHEST,
                        preferred_element_type=jnp.float32)
    h = h + b_ref[...]
    o_ref[...] = _bn_gelu(h, g_ref[...], be_ref[...])


def _tc_scale_body(h_ref, degp_ref, hp_ref, dis_ref):
    deg = degp_ref[0] + degp_ref[1]
    dis = lax.rsqrt(deg)
    dis_ref[...] = dis
    hp_ref[...] = h_ref[...] * dis


def _tc_layer2_body(acc_ref, dis_ref, wt_ref, b_ref, g_ref, be_ref, o_ref):
    dis = dis_ref[...]
    a = (acc_ref[0] + acc_ref[1]) * dis
    h = lax.dot_general(a, wt_ref[...], (((1,), (0,)), ((), ())),
                        precision=lax.Precision.HIGHEST,
                        preferred_element_type=jnp.float32)
    h = h + b_ref[...]
    o_ref[...] = _bn_gelu(h, g_ref[...], be_ref[...]) * dis


def _tc_final_body(acc_ref, dis_ref, o_ref):
    o_ref[...] = (acc_ref[0] + acc_ref[1]) * dis_ref[...]


def _row_spec(r, cdim):
    return pl.BlockSpec((r, cdim), lambda i: (i, 0))


def _full_spec(shape):
    return pl.BlockSpec(shape, lambda i: tuple(0 for _ in shape))


def _tc_dense1(x, w1t, b1, g1, be1):
    return pl.pallas_call(
        _tc_dense1_body,
        grid=(N // _BLK,),
        in_specs=[_row_spec(_BLK, D), _full_spec((D, D)), _full_spec((1, D)),
                  _row_spec(_BLK, 1), _row_spec(_BLK, 1)],
        out_specs=_row_spec(_BLK, D),
        out_shape=jax.ShapeDtypeStruct((N, D), jnp.float32),
    )(x, w1t, b1, g1, be1)


def _tc_scale(h, degp):
    return pl.pallas_call(
        _tc_scale_body,
        grid=(N // _BLK,),
        in_specs=[_row_spec(_BLK, D),
                  pl.BlockSpec((NCORES, _BLK, 1), lambda i: (0, i, 0))],
        out_specs=[_row_spec(_BLK, D), _row_spec(_BLK, 1)],
        out_shape=[jax.ShapeDtypeStruct((N, D), jnp.float32),
                   jax.ShapeDtypeStruct((N, 1), jnp.float32)],
    )(h, degp)


def _tc_layer2(accp, dis, w2t, b2, g2, be2):
    return pl.pallas_call(
        _tc_layer2_body,
        grid=(N // _BLK,),
        in_specs=[pl.BlockSpec((NCORES, _BLK, D), lambda i: (0, i, 0)),
                  _row_spec(_BLK, 1), _full_spec((D, D)), _full_spec((1, D)),
                  _row_spec(_BLK, 1), _row_spec(_BLK, 1)],
        out_specs=_row_spec(_BLK, D),
        out_shape=jax.ShapeDtypeStruct((N, D), jnp.float32),
    )(accp, dis, w2t, b2, g2, be2)


def _tc_final(accp, dis):
    return pl.pallas_call(
        _tc_final_body,
        grid=(N // _BLK,),
        in_specs=[pl.BlockSpec((NCORES, _BLK, D), lambda i: (0, i, 0)),
                  _row_spec(_BLK, 1)],
        out_specs=_row_spec(_BLK, D),
        out_shape=jax.ShapeDtypeStruct((N, D), jnp.float32),
    )(accp, dis)


def kernel(x, W1, b1, g1, be1, W2, b2, g2, be2, edge_index):
    xs = x[0]
    row = edge_index[0]
    col = edge_index[1]
    zeros2d = jnp.zeros((W, D), jnp.float32)
    zeros1d = jnp.zeros((N,), jnp.float32)
    ones1d = jnp.ones((W,), jnp.float32)

    degp = _sc_degree(col, zeros1d, ones1d)
    h1 = _tc_dense1(xs, W1.T, b1.reshape(1, D), g1.reshape(N, 1),
                    be1.reshape(N, 1))
    h1p, dis = _tc_scale(h1, degp.reshape(NCORES, N, 1))
    accp1 = _sc_mp(h1p, row, col, zeros2d)
    h2p = _tc_layer2(accp1, dis, W2.T, b2.reshape(1, D), g2.reshape(N, 1),
                     be2.reshape(N, 1))
    accp2 = _sc_mp(h2p, row, col, zeros2d)
    out = _tc_final(accp2, dis)
    return out[None]


# speedup vs baseline: 22.1103x; 1.0793x over previous
import functools

import jax
import jax.numpy as jnp
from jax import lax
from jax.experimental import pallas as pl
from jax.experimental.pallas import tpu as pltpu
from jax.experimental.pallas import tpu_sc as plsc

N = 10000
E = 320000
D = 128
EPS = 1e-5

W = 128
NWIN = E // W
NCORES = 2
NSUB = 16
NWORK = NCORES * NSUB
RCHUNK = 80
NCHUNK = N // RCHUNK

_MESH = plsc.VectorSubcoreMesh(core_axis_name="c", subcore_axis_name="s")


@functools.partial(
    pl.kernel,
    out_type=jax.ShapeDtypeStruct((NCORES, N), jnp.float32),
    mesh=_MESH,
    scratch_types=[
        pltpu.VMEM_SHARED((N,), jnp.float32),
        pltpu.VMEM((N,), jnp.float32),
        pltpu.VMEM((W,), jnp.float32),
        pltpu.VMEM((2, W), jnp.int32),
        pltpu.SemaphoreType.DMA,
        pltpu.SemaphoreType.DMA,
    ],
)
def _sc_degree(col_hbm, zeros_hbm, ones_hbm, deg_hbm, dacc, zbuf, ones_v,
               cidx2, isem0, isem1):
    c = lax.axis_index("c")
    s = lax.axis_index("s")
    w = c * NSUB + s

    @pl.when(s == 0)
    def _():
        pltpu.sync_copy(zeros_hbm, zbuf)
        pltpu.sync_copy(zbuf, dacc)
    pltpu.sync_copy(ones_hbm, ones_v)
    plsc.subcore_barrier()

    nfull = NWIN // NWORK
    npair = nfull // 2

    def eb(j):
        return (w + j * NWORK) * W

    pltpu.sync_copy(col_hbm.at[pl.ds(eb(0), W)], cidx2.at[0])

    @pl.loop(0, npair)
    def _(r):
        pltpu.async_copy(col_hbm.at[pl.ds(eb(2 * r + 1), W)], cidx2.at[1],
                         isem1)
        pltpu.sync_copy(ones_v, dacc.at[cidx2.at[0]], add=True)

        @pl.when(r < npair - 1)
        def _():
            pltpu.async_copy(col_hbm.at[pl.ds(eb(2 * r + 2), W)], cidx2.at[0],
                             isem0)

        pltpu.make_async_copy(col_hbm.at[pl.ds(0, W)], cidx2.at[1],
                              isem1).wait()
        pltpu.sync_copy(ones_v, dacc.at[cidx2.at[1]], add=True)

        @pl.when(r < npair - 1)
        def _():
            pltpu.make_async_copy(col_hbm.at[pl.ds(0, W)], cidx2.at[0],
                                  isem0).wait()

    @pl.when(w < NWIN - nfull * NWORK)
    def _():
        pltpu.sync_copy(col_hbm.at[pl.ds((nfull * NWORK + w) * W, W)],
                        cidx2.at[0])
        pltpu.sync_copy(ones_v, dacc.at[cidx2.at[0]], add=True)

    plsc.subcore_barrier()

    @pl.when(s == 0)
    def _():
        pltpu.sync_copy(dacc, deg_hbm.at[c])


WPW = 80
IB = 8
NBATCH = WPW // IB
NPAD = N + 16


@functools.partial(
    pl.kernel,
    out_type=jax.ShapeDtypeStruct((NCORES, NPAD, D), jnp.float32),
    mesh=_MESH,
    scratch_types=[
        pltpu.VMEM_SHARED((NPAD, D), jnp.float32),
        pltpu.VMEM((2, W, D), jnp.float32),
        pltpu.VMEM((2, IB, W), jnp.int32),
        pltpu.VMEM((2, IB, W), jnp.int32),
        pltpu.SemaphoreType.DMA,
        pltpu.SemaphoreType.DMA,
        pltpu.SemaphoreType.DMA,
        pltpu.SemaphoreType.DMA,
        pltpu.SemaphoreType.DMA,
    ],
)
def _sc_mp(h_hbm, row_hbm, col_hbm, zeros_hbm, out_hbm, acc, rows2, cidxb,
           ridxb, g0, g1, s0, s1, isem):
    gsem = [g0, g1]
    ssem = [s0, s1]
    c = lax.axis_index("c")
    s = lax.axis_index("s")
    w = c * NSUB + s
    wb = w * WPW

    pltpu.sync_copy(zeros_hbm, rows2.at[0])

    @pl.loop(0, (NCHUNK + NSUB - 1) // NSUB)
    def _(t):
        ck = s + t * NSUB

        @pl.when(ck < NCHUNK)
        def _():
            pltpu.sync_copy(rows2.at[0, pl.ds(0, RCHUNK)],
                            acc.at[pl.ds(ck * RCHUNK, RCHUNK)])

    @pl.when(s == 0)
    def _():
        pltpu.sync_copy(rows2.at[0, pl.ds(0, NPAD - N)],
                        acc.at[pl.ds(N, NPAD - N)])

    plsc.subcore_barrier()

    def load_batch_idx(kdyn, pp):
        pltpu.async_copy(col_hbm.at[pl.ds(wb + kdyn * IB, IB)],
                         cidxb.at[pp], isem)
        pltpu.async_copy(row_hbm.at[pl.ds(wb + kdyn * IB, IB)],
                         ridxb.at[pp], isem)

    def wait_batch_idx(kdyn, pp):
        pltpu.make_async_copy(col_hbm.at[pl.ds(wb + kdyn * IB, IB)],
                              cidxb.at[pp], isem).wait()
        pltpu.make_async_copy(row_hbm.at[pl.ds(wb + kdyn * IB, IB)],
                              ridxb.at[pp], isem).wait()

    def start_gather(pp, i, b):
        pltpu.async_copy(h_hbm.at[cidxb.at[pp, i]], rows2.at[b], gsem[b])

    def wait_gather(pp, i, b):
        pltpu.make_async_copy(h_hbm.at[cidxb.at[pp, i]], rows2.at[b],
                              gsem[b]).wait()

    def start_scatter(b, pp, i):
        pltpu.async_copy(rows2.at[b], acc.at[ridxb.at[pp, i]], ssem[b],
                         add=True)

    def wait_scatter(b, pp, i):
        pltpu.make_async_copy(rows2.at[b], acc.at[ridxb.at[pp, i]],
                              ssem[b]).wait()

    pltpu.sync_copy(col_hbm.at[pl.ds(wb, IB)], cidxb.at[0])
    pltpu.sync_copy(row_hbm.at[pl.ds(wb, IB)], ridxb.at[0])
    start_gather(0, 0, 0)

    @pl.loop(0, NBATCH // 2)
    def _(m):
        for kk in range(2):
            k = 2 * m + kk
            P = kk
            PN = 1 - kk
            for i in range(IB):
                b = i % 2

                wait_gather(P, i, b)
                start_scatter(b, P, i)

                if i == 2:
                    if kk == 0:
                        load_batch_idx(k + 1, PN)
                    else:
                        @pl.when(m < NBATCH // 2 - 1)
                        def _(k=k, PN=PN):
                            load_batch_idx(k + 1, PN)

                if i < IB - 1:
                    if i == 0:
                        if kk == 0:
                            @pl.when(m > 0)
                            def _(PN=PN):
                                wait_scatter(1, PN, IB - 1)
                        else:
                            wait_scatter(1, PN, IB - 1)
                    else:
                        wait_scatter(1 - b, P, i - 1)
                    start_gather(P, i + 1, 1 - b)
                else:
                    def cross(b=b, P=P, PN=PN, k=k, i=i):
                        wait_scatter(1 - b, P, i - 1)
                        wait_batch_idx(k + 1, PN)
                        start_gather(PN, 0, 1 - b)
                    if kk == 0:
                        cross()
                    else:
                        @pl.when(m < NBATCH // 2 - 1)
                        def _(cross=cross):
                            cross()

    wait_scatter(0, 1, IB - 2)
    wait_scatter(1, 1, IB - 1)

    plsc.subcore_barrier()

    @pl.loop(0, (NCHUNK + NSUB - 1) // NSUB)
    def _(t):
        ck = s + t * NSUB

        @pl.when(ck < NCHUNK)
        def _():
            pltpu.sync_copy(acc.at[pl.ds(ck * RCHUNK, RCHUNK)],
                            out_hbm.at[c, pl.ds(ck * RCHUNK, RCHUNK)])


_BLK = 2000


def _bn_gelu(h, g, be):
    mu = jnp.mean(h, axis=1, keepdims=True)
    var = jnp.mean((h - mu) ** 2, axis=1, keepdims=True)
    h = (h - mu) * lax.rsqrt(var + EPS)
    h = h * g + be
    return 0.5 * h * (1.0 + lax.erf(h * (2.0 ** -0.5)))


def _tc_dense1_body(x_ref, wt_ref, b_ref, g_ref, be_ref, o_ref):
    h = lax.dot_general(x_ref[...], wt_ref[...], (((1,), (0,)), ((), ())),
                        precision=lax.Precision.HIGHEST,
                        preferred_element_type=jnp.float32)
    h = h + b_ref[...]
    o_ref[...] = _bn_gelu(h, g_ref[...], be_ref[...])


def _tc_scale_body(h_ref, degp_ref, hp_ref, dis_ref):
    deg = degp_ref[0] + degp_ref[1]
    dis = lax.rsqrt(deg)
    dis_ref[...] = dis
    hp_ref[...] = h_ref[...] * dis


def _tc_layer2_body(acc_ref, dis_ref, wt_ref, b_ref, g_ref, be_ref, o_ref):
    dis = dis_ref[...]
    a = (acc_ref[0] + acc_ref[1]) * dis
    h = lax.dot_general(a, wt_ref[...], (((1,), (0,)), ((), ())),
                        precision=lax.Precision.HIGHEST,
                        preferred_element_type=jnp.float32)
    h = h + b_ref[...]
    o_ref[...] = _bn_gelu(h, g_ref[...], be_ref[...]) * dis


def _tc_final_body(acc_ref, dis_ref, o_ref):
    o_ref[...] = (acc_ref[0] + acc_ref[1]) * dis_ref[...]


def _row_spec(r, cdim):
    return pl.BlockSpec((r, cdim), lambda i: (i, 0))


def _full_spec(shape):
    return pl.BlockSpec(shape, lambda i: tuple(0 for _ in shape))


def _tc_dense1(x, w1t, b1, g1, be1):
    return pl.pallas_call(
        _tc_dense1_body,
        grid=(N // _BLK,),
        in_specs=[_row_spec(_BLK, D), _full_spec((D, D)), _full_spec((1, D)),
                  _row_spec(_BLK, 1), _row_spec(_BLK, 1)],
        out_specs=_row_spec(_BLK, D),
        out_shape=jax.ShapeDtypeStruct((N, D), jnp.float32),
    )(x, w1t, b1, g1, be1)


def _tc_scale(h, degp):
    return pl.pallas_call(
        _tc_scale_body,
        grid=(N // _BLK,),
        in_specs=[_row_spec(_BLK, D),
                  pl.BlockSpec((NCORES, _BLK, 1), lambda i: (0, i, 0))],
        out_specs=[_row_spec(_BLK, D), _row_spec(_BLK, 1)],
        out_shape=[jax.ShapeDtypeStruct((N, D), jnp.float32),
                   jax.ShapeDtypeStruct((N, 1), jnp.float32)],
    )(h, degp)


def _tc_layer2(accp, dis, w2t, b2, g2, be2):
    return pl.pallas_call(
        _tc_layer2_body,
        grid=(N // _BLK,),
        in_specs=[pl.BlockSpec((NCORES, _BLK, D), lambda i: (0, i, 0)),
                  _row_spec(_BLK, 1), _full_spec((D, D)), _full_spec((1, D)),
                  _row_spec(_BLK, 1), _row_spec(_BLK, 1)],
        out_specs=_row_spec(_BLK, D),
        out_shape=jax.ShapeDtypeStruct((N, D), jnp.float32),
    )(accp, dis, w2t, b2, g2, be2)


def _tc_final(accp, dis):
    return pl.pallas_call(
        _tc_final_body,
        grid=(N // _BLK,),
        in_specs=[pl.BlockSpec((NCORES, _BLK, D), lambda i: (0, i, 0)),
                  _row_spec(_BLK, 1)],
        out_specs=_row_spec(_BLK, D),
        out_shape=jax.ShapeDtypeStruct((N, D), jnp.float32),
    )(accp, dis)


def _window_layout(idx, pad_vals):
    nfull = NWIN // NWORK
    win = idx.reshape(NWIN, W)
    main = win[: NWORK * nfull].reshape(NWORK, nfull, W)
    pad = jnp.broadcast_to(pad_vals, (NWORK, 1, W)).astype(jnp.int32)
    extra = jnp.concatenate(
        [win[NWORK * nfull:],
         jnp.broadcast_to(pad_vals, (NWORK - (NWIN - NWORK * nfull), W))],
        axis=0).astype(jnp.int32)[:, None, :]
    return jnp.concatenate([main, extra, pad], axis=1).reshape(NWORK * WPW, W)


def kernel(x, W1, b1, g1, be1, W2, b2, g2, be2, edge_index):
    xs = x[0]
    row = edge_index[0]
    col = edge_index[1]
    zeros2d = jnp.zeros((W, D), jnp.float32)
    zeros1d = jnp.zeros((N,), jnp.float32)
    ones1d = jnp.ones((W,), jnp.float32)
    lane = jnp.arange(W, dtype=jnp.int32)
    col_l = _window_layout(col, lane)
    row_l = _window_layout(row, N + (lane % (NPAD - N)))

    degp = _sc_degree(col, zeros1d, ones1d)
    h1 = _tc_dense1(xs, W1.T, b1.reshape(1, D), g1.reshape(N, 1),
                    be1.reshape(N, 1))
    h1p, dis = _tc_scale(h1, degp.reshape(NCORES, N, 1))
    accp1 = _sc_mp(h1p, row_l, col_l, zeros2d)
    h2p = _tc_layer2(accp1, dis, W2.T, b2.reshape(1, D), g2.reshape(N, 1),
                     be2.reshape(N, 1))
    accp2 = _sc_mp(h2p, row_l, col_l, zeros2d)
    out = _tc_final(accp2, dis)
    return out[None]
